# baseline (speedup 1.0000x reference)
import functools

import jax
import jax.numpy as jnp
from jax import lax
from jax.experimental import pallas as pl
from jax.experimental.pallas import tpu as pltpu
from jax.experimental.pallas import tpu_sc as plsc

BATCH = 16384
EMB = 64
GLOBAL_MEAN = 3.5

NUM_CORES = 2
NUM_SUBCORES = 16
NW = NUM_CORES * NUM_SUBCORES
PER_W = BATCH // NW
CHUNK = 128
NCHUNK = PER_W // CHUNK
LANES = 16
GPC = CHUNK // LANES


NCH = PER_W // LANES


def _dots_body(uidx_hbm, midx_hbm, uemb3_hbm, memb3_hbm, part_hbm, dots_hbm,
               idxu_v, idxm_v, part_v, ug0, ug1, mg0, mg1,
               ubuf0, ubuf1, mbuf0, mbuf1, dots_v, shared, semu, semm):
    sid = lax.axis_index("s")
    co = lax.axis_index("c")
    wid = co * NUM_SUBCORES + sid
    pairbase = (wid // 2) * (2 * PER_W)
    ioff = (wid % 2) * PER_W
    pltpu.sync_copy(uidx_hbm.at[pl.ds(pairbase, 2 * PER_W)], idxu_v)
    pltpu.sync_copy(midx_hbm.at[pl.ds(pairbase, 2 * PER_W)], idxm_v)
    pltpu.sync_copy(part_hbm.at[pl.ds(pairbase, 2 * PER_W)], part_v)

    lanes = lax.iota(jnp.int32, LANES)
    ubufs = (ubuf0, ubuf1)
    mbufs = (mbuf0, mbuf1)
    ugs = (ug0, ug1)
    mgs = (mg0, mg1)

    def prep_fire(c, p):
        sl = pl.ds(ioff + c * LANES, LANES)
        urb = lax.shift_right_logical(idxu_v[sl], 3)
        mrb = lax.shift_right_logical(idxm_v[sl], 3)
        for b in range(LANES):
            pltpu.async_copy(uemb3_hbm.at[urb[b]], ubufs[p].at[b], semu)
            pltpu.async_copy(memb3_hbm.at[mrb[b]], mbufs[p].at[b], semm)

    def wait(p):
        for b in range(LANES):
            pltpu.make_async_copy(uemb3_hbm.at[0], ubufs[p].at[b], semu).wait()
            pltpu.make_async_copy(memb3_hbm.at[0], mbufs[p].at[b], semm).wait()

    def compute(c, p):
        uvals = idxu_v[pl.ds(ioff + c * LANES, LANES)]
        mvals = idxm_v[pl.ds(ioff + c * LANES, LANES)]
        usub = uvals & 7
        msub = mvals & 7
        acc = jnp.zeros((LANES,), jnp.float32)
        for d in range(EMB):
            rotd = (d + lanes) & (EMB - 1)
            u = plsc.load_gather(ubufs[p], [lanes, usub, rotd])
            m = plsc.load_gather(mbufs[p], [lanes, msub, rotd])
            acc = acc + u * m
        acc = acc + part_v[pl.ds(ioff + c * LANES, LANES)]
        plsc.store_scatter(dots_v, [c * LANES + lanes], acc)

    prep_fire(0, 0)

    def pair(t, carry):
        prep_fire(2 * t + 1, 1)
        wait(0)
        compute(2 * t, 0)

        @pl.when(t < NCH // 2 - 1)
        def _():
            prep_fire(2 * t + 2, 0)
        wait(1)
        compute(2 * t + 1, 1)
        return carry

    lax.fori_loop(0, NCH // 2, pair, 0)

    pltpu.sync_copy(dots_v, shared.at[sid])
    plsc.subcore_barrier()

    @pl.when(sid == 0)
    def _():
        pltpu.sync_copy(shared, dots_hbm.at[pl.ds(co * NUM_SUBCORES, NUM_SUBCORES), :])


_dots_call = functools.partial(
    pl.kernel,
    out_type=jax.ShapeDtypeStruct((NW, PER_W), jnp.float32),
    mesh=plsc.VectorSubcoreMesh(core_axis_name="c", subcore_axis_name="s"),
    compiler_params=pltpu.CompilerParams(
        needs_layout_passes=False, use_tc_tiling_on_sc=True),
    scratch_types=[
        pltpu.VMEM((2 * PER_W,), jnp.int32),
        pltpu.VMEM((2 * PER_W,), jnp.int32),
        pltpu.VMEM((2 * PER_W,), jnp.float32),
        pltpu.VMEM((LANES,), jnp.int32),
        pltpu.VMEM((LANES,), jnp.int32),
        pltpu.VMEM((LANES,), jnp.int32),
        pltpu.VMEM((LANES,), jnp.int32),
        pltpu.VMEM((LANES, 8, EMB), jnp.float32),
        pltpu.VMEM((LANES, 8, EMB), jnp.float32),
        pltpu.VMEM((LANES, 8, EMB), jnp.float32),
        pltpu.VMEM((LANES, 8, EMB), jnp.float32),
        pltpu.VMEM((PER_W,), jnp.float32),
        pltpu.VMEM_SHARED((NUM_SUBCORES, PER_W), jnp.float32),
        pltpu.SemaphoreType.DMA,
        pltpu.SemaphoreType.DMA,
    ],
)(_dots_body)


def _bias_body(uidx_hbm, midx_hbm, ub16_hbm, mb16_hbm,
               out_hbm, uidx_v, midx_v, uhi_v, mhi_v, ubrows,
               mbrows, out_v, sem):
    sid = lax.axis_index("s")
    co = lax.axis_index("c")
    wid = co * NUM_SUBCORES + sid
    pltpu.sync_copy(uidx_hbm.at[wid], uidx_v)
    pltpu.sync_copy(midx_hbm.at[wid], midx_v)

    for t in range(PER_W // LANES):
        tsl = pl.ds(t * LANES, LANES)
        uhi_v[tsl] = lax.shift_right_logical(uidx_v[tsl], 4)
        mhi_v[tsl] = lax.shift_right_logical(midx_v[tsl], 4)
    copies = []
    for j in range(NCHUNK):
        isl = pl.ds(j * CHUNK, CHUNK)
        copies.append(pltpu.async_copy(ub16_hbm.at[uhi_v.at[isl]], ubrows.at[isl, :], sem))
        copies.append(pltpu.async_copy(mb16_hbm.at[mhi_v.at[isl]], mbrows.at[isl, :], sem))
    for c in copies:
        c.wait()

    lanes = lax.iota(jnp.int32, LANES)

    def group(g, carry):
        ib = g * LANES
        rvec = ib + lanes
        uld = uidx_v[pl.ds(ib, LANES)]
        mld = midx_v[pl.ds(ib, LANES)]
        ub = plsc.load_gather(ubrows, [rvec, uld & 15])
        mb = plsc.load_gather(mbrows, [rvec, mld & 15])
        res = ub + mb + jnp.float32(GLOBAL_MEAN)
        plsc.store_scatter(out_v, [ib + lanes], res)
        return carry

    lax.fori_loop(0, PER_W // LANES, group, 0)
    pltpu.sync_copy(out_v, out_hbm.at[wid])


_bias_call = functools.partial(
    pl.kernel,
    out_type=jax.ShapeDtypeStruct((NW, PER_W), jnp.float32),
    mesh=plsc.VectorSubcoreMesh(core_axis_name="c", subcore_axis_name="s"),
    compiler_params=pltpu.CompilerParams(
        needs_layout_passes=False, use_tc_tiling_on_sc=False),
    scratch_types=[
        pltpu.VMEM((PER_W,), jnp.int32),
        pltpu.VMEM((PER_W,), jnp.int32),
        pltpu.VMEM((PER_W,), jnp.int32),
        pltpu.VMEM((PER_W,), jnp.int32),
        pltpu.VMEM((PER_W, LANES), jnp.float32),
        pltpu.VMEM((PER_W, LANES), jnp.float32),
        pltpu.VMEM((PER_W,), jnp.float32),
        pltpu.SemaphoreType.DMA,
    ],
)(_bias_body)


def kernel(user_idx, movie_idx, user_emb, movie_emb, user_bias, movie_bias):
    uidx = user_idx.astype(jnp.int32)
    midx = movie_idx.astype(jnp.int32)
    uemb3 = user_emb.reshape(-1, 8, EMB)
    memb3 = movie_emb.reshape(-1, 8, EMB)
    part = _bias_call(uidx.reshape(NW, PER_W), midx.reshape(NW, PER_W),
                      user_bias.reshape(-1, LANES),
                      movie_bias.reshape(-1, LANES))
    out = _dots_call(uidx, midx, uemb3, memb3, part.reshape(BATCH))
    return out.reshape(BATCH)


# speedup vs baseline: 1.0065x; 1.0065x over previous
import functools

import jax
import jax.numpy as jnp
from jax import lax
from jax.experimental import pallas as pl
from jax.experimental.pallas import tpu as pltpu
from jax.experimental.pallas import tpu_sc as plsc

BATCH = 16384
EMB = 64
GLOBAL_MEAN = 3.5

NUM_CORES = 2
NUM_SUBCORES = 16
NW = NUM_CORES * NUM_SUBCORES
PER_W = BATCH // NW
CHUNK = 128
NCHUNK = PER_W // CHUNK
LANES = 16
GPC = CHUNK // LANES


NCH = PER_W // LANES


def _dots_body(uidx_hbm, midx_hbm, uemb3_hbm, memb3_hbm, dots_hbm,
               idxu_v, idxm_v, ug0, ug1, mg0, mg1,
               ubuf0, ubuf1, mbuf0, mbuf1, dots_v, shared, semu, semm):
    sid = lax.axis_index("s")
    co = lax.axis_index("c")
    wid = co * NUM_SUBCORES + sid
    pairbase = (wid // 2) * (2 * PER_W)
    ioff = (wid % 2) * PER_W
    pltpu.sync_copy(uidx_hbm.at[pl.ds(pairbase, 2 * PER_W)], idxu_v)
    pltpu.sync_copy(midx_hbm.at[pl.ds(pairbase, 2 * PER_W)], idxm_v)

    lanes = lax.iota(jnp.int32, LANES)
    ubufs = (ubuf0, ubuf1)
    mbufs = (mbuf0, mbuf1)
    ugs = (ug0, ug1)
    mgs = (mg0, mg1)

    def prep_fire(c, p):
        sl = pl.ds(ioff + c * LANES, LANES)
        urb = lax.shift_right_logical(idxu_v[sl], 3)
        mrb = lax.shift_right_logical(idxm_v[sl], 3)
        for b in range(LANES):
            pltpu.async_copy(uemb3_hbm.at[urb[b]], ubufs[p].at[b], semu)
            pltpu.async_copy(memb3_hbm.at[mrb[b]], mbufs[p].at[b], semm)

    def wait(p):
        for b in range(LANES):
            pltpu.make_async_copy(uemb3_hbm.at[0], ubufs[p].at[b], semu).wait()
            pltpu.make_async_copy(memb3_hbm.at[0], mbufs[p].at[b], semm).wait()

    def compute(c, p):
        uvals = idxu_v[pl.ds(ioff + c * LANES, LANES)]
        mvals = idxm_v[pl.ds(ioff + c * LANES, LANES)]
        usub = uvals & 7
        msub = mvals & 7
        acc = jnp.zeros((LANES,), jnp.float32)
        for d in range(EMB):
            rotd = (d + lanes) & (EMB - 1)
            u = plsc.load_gather(ubufs[p], [lanes, usub, rotd])
            m = plsc.load_gather(mbufs[p], [lanes, msub, rotd])
            acc = acc + u * m
        plsc.store_scatter(dots_v, [c * LANES + lanes], acc)

    prep_fire(0, 0)

    def pair(t, carry):
        prep_fire(2 * t + 1, 1)
        wait(0)
        compute(2 * t, 0)

        @pl.when(t < NCH // 2 - 1)
        def _():
            prep_fire(2 * t + 2, 0)
        wait(1)
        compute(2 * t + 1, 1)
        return carry

    lax.fori_loop(0, NCH // 2, pair, 0)

    pltpu.sync_copy(dots_v, shared.at[sid])
    plsc.subcore_barrier()

    @pl.when(sid == 0)
    def _():
        pltpu.sync_copy(shared, dots_hbm.at[pl.ds(co * NUM_SUBCORES, NUM_SUBCORES), :])


_dots_call = functools.partial(
    pl.kernel,
    out_type=jax.ShapeDtypeStruct((NW, PER_W), jnp.float32),
    mesh=plsc.VectorSubcoreMesh(core_axis_name="c", subcore_axis_name="s"),
    compiler_params=pltpu.CompilerParams(
        needs_layout_passes=False, use_tc_tiling_on_sc=True),
    scratch_types=[
        pltpu.VMEM((2 * PER_W,), jnp.int32),
        pltpu.VMEM((2 * PER_W,), jnp.int32),
        pltpu.VMEM((LANES,), jnp.int32),
        pltpu.VMEM((LANES,), jnp.int32),
        pltpu.VMEM((LANES,), jnp.int32),
        pltpu.VMEM((LANES,), jnp.int32),
        pltpu.VMEM((LANES, 8, EMB), jnp.float32),
        pltpu.VMEM((LANES, 8, EMB), jnp.float32),
        pltpu.VMEM((LANES, 8, EMB), jnp.float32),
        pltpu.VMEM((LANES, 8, EMB), jnp.float32),
        pltpu.VMEM((PER_W,), jnp.float32),
        pltpu.VMEM_SHARED((NUM_SUBCORES, PER_W), jnp.float32),
        pltpu.SemaphoreType.DMA,
        pltpu.SemaphoreType.DMA,
    ],
)(_dots_body)


def _final_body(uidx_hbm, midx_hbm, dots_hbm, ub16_hbm, mb16_hbm,
                out_hbm, uidx_v, midx_v, uhi_v, mhi_v, dvals_v, ubrows,
                mbrows, out_v, sem):
    sid = lax.axis_index("s")
    co = lax.axis_index("c")
    wid = co * NUM_SUBCORES + sid
    pltpu.sync_copy(uidx_hbm.at[wid], uidx_v)
    pltpu.sync_copy(midx_hbm.at[wid], midx_v)
    pltpu.sync_copy(dots_hbm.at[wid], dvals_v)

    for t in range(PER_W // LANES):
        tsl = pl.ds(t * LANES, LANES)
        uhi_v[tsl] = lax.shift_right_logical(uidx_v[tsl], 4)
        mhi_v[tsl] = lax.shift_right_logical(midx_v[tsl], 4)
    copies = []
    for j in range(NCHUNK):
        isl = pl.ds(j * CHUNK, CHUNK)
        copies.append(pltpu.async_copy(ub16_hbm.at[uhi_v.at[isl]], ubrows.at[isl, :], sem))
        copies.append(pltpu.async_copy(mb16_hbm.at[mhi_v.at[isl]], mbrows.at[isl, :], sem))
    for c in copies:
        c.wait()

    lanes = lax.iota(jnp.int32, LANES)

    def group(g, carry):
        ib = g * LANES
        rvec = ib + lanes
        uld = uidx_v[pl.ds(ib, LANES)]
        mld = midx_v[pl.ds(ib, LANES)]
        dv = dvals_v[pl.ds(ib, LANES)]
        ub = plsc.load_gather(ubrows, [rvec, uld & 15])
        mb = plsc.load_gather(mbrows, [rvec, mld & 15])
        res = dv + ub + mb + jnp.float32(GLOBAL_MEAN)
        plsc.store_scatter(out_v, [ib + lanes], res)
        return carry

    lax.fori_loop(0, PER_W // LANES, group, 0)
    pltpu.sync_copy(out_v, out_hbm.at[wid])


_final_call = functools.partial(
    pl.kernel,
    out_type=jax.ShapeDtypeStruct((NW, PER_W), jnp.float32),
    mesh=plsc.VectorSubcoreMesh(core_axis_name="c", subcore_axis_name="s"),
    compiler_params=pltpu.CompilerParams(
        needs_layout_passes=False, use_tc_tiling_on_sc=False),
    scratch_types=[
        pltpu.VMEM((PER_W,), jnp.int32),
        pltpu.VMEM((PER_W,), jnp.int32),
        pltpu.VMEM((PER_W,), jnp.int32),
        pltpu.VMEM((PER_W,), jnp.int32),
        pltpu.VMEM((PER_W,), jnp.float32),
        pltpu.VMEM((PER_W, LANES), jnp.float32),
        pltpu.VMEM((PER_W, LANES), jnp.float32),
        pltpu.VMEM((PER_W,), jnp.float32),
        pltpu.SemaphoreType.DMA,
    ],
)(_final_body)


def kernel(user_idx, movie_idx, user_emb, movie_emb, user_bias, movie_bias):
    uidx = user_idx.astype(jnp.int32)
    midx = movie_idx.astype(jnp.int32)
    uemb3 = user_emb.reshape(-1, 8, EMB)
    memb3 = movie_emb.reshape(-1, 8, EMB)
    dots = _dots_call(uidx, midx, uemb3, memb3)
    out = _final_call(uidx.reshape(NW, PER_W), midx.reshape(NW, PER_W),
                      dots, user_bias.reshape(-1, LANES),
                      movie_bias.reshape(-1, LANES))
    return out.reshape(BATCH)


# speedup vs baseline: 1.0081x; 1.0016x over previous
import functools

import jax
import jax.numpy as jnp
from jax import lax
from jax.experimental import pallas as pl
from jax.experimental.pallas import tpu as pltpu
from jax.experimental.pallas import tpu_sc as plsc

BATCH = 16384
EMB = 64
GLOBAL_MEAN = 3.5

NUM_CORES = 2
NUM_SUBCORES = 16
NW = NUM_CORES * NUM_SUBCORES
PER_W = BATCH // NW
CHUNK = 128
NCHUNK = PER_W // CHUNK
LANES = 16
GPC = CHUNK // LANES


NCH = PER_W // LANES


def _dots_body(uidx_hbm, midx_hbm, uemb3_hbm, memb3_hbm, dots_hbm,
               idxu_v, idxm_v,
               ubuf0, ubuf1, ubuf2, mbuf0, mbuf1, mbuf2,
               dots_v, shared, semu, semm):
    sid = lax.axis_index("s")
    co = lax.axis_index("c")
    wid = co * NUM_SUBCORES + sid
    pairbase = (wid // 2) * (2 * PER_W)
    ioff = (wid % 2) * PER_W
    pltpu.sync_copy(uidx_hbm.at[pl.ds(pairbase, 2 * PER_W)], idxu_v)
    pltpu.sync_copy(midx_hbm.at[pl.ds(pairbase, 2 * PER_W)], idxm_v)

    lanes = lax.iota(jnp.int32, LANES)
    ubufs = (ubuf0, ubuf1, ubuf2)
    mbufs = (mbuf0, mbuf1, mbuf2)

    def prep_fire(c, p):
        sl = pl.ds(ioff + c * LANES, LANES)
        urb = lax.shift_right_logical(idxu_v[sl], 3)
        mrb = lax.shift_right_logical(idxm_v[sl], 3)
        for b in range(LANES):
            pltpu.async_copy(uemb3_hbm.at[urb[b]], ubufs[p].at[b], semu)
            pltpu.async_copy(memb3_hbm.at[mrb[b]], mbufs[p].at[b], semm)

    def wait(p):
        for b in range(LANES):
            pltpu.make_async_copy(uemb3_hbm.at[0], ubufs[p].at[b], semu).wait()
            pltpu.make_async_copy(memb3_hbm.at[0], mbufs[p].at[b], semm).wait()

    def compute(c, p):
        uvals = idxu_v[pl.ds(ioff + c * LANES, LANES)]
        mvals = idxm_v[pl.ds(ioff + c * LANES, LANES)]
        usub = uvals & 7
        msub = mvals & 7
        acc = jnp.zeros((LANES,), jnp.float32)
        for d in range(EMB):
            rotd = (d + lanes) & (EMB - 1)
            u = plsc.load_gather(ubufs[p], [lanes, usub, rotd])
            m = plsc.load_gather(mbufs[p], [lanes, msub, rotd])
            acc = acc + u * m
        plsc.store_scatter(dots_v, [c * LANES + lanes], acc)

    prep_fire(0, 0)
    prep_fire(1, 1)

    def triple(t, carry):
        for k in range(3):
            c = 3 * t + k
            prep_fire(c + 2, (k + 2) % 3)
            wait(k)
            compute(c, k)
        return carry

    lax.fori_loop(0, (NCH - 2) // 3, triple, 0)
    for c, k in ((NCH - 2, 0), (NCH - 1, 1)):
        wait(k)
        compute(c, k)

    pltpu.sync_copy(dots_v, shared.at[sid])
    plsc.subcore_barrier()

    @pl.when(sid == 0)
    def _():
        pltpu.sync_copy(shared, dots_hbm.at[pl.ds(co * NUM_SUBCORES, NUM_SUBCORES), :])


_dots_call = functools.partial(
    pl.kernel,
    out_type=jax.ShapeDtypeStruct((NW, PER_W), jnp.float32),
    mesh=plsc.VectorSubcoreMesh(core_axis_name="c", subcore_axis_name="s"),
    compiler_params=pltpu.CompilerParams(
        needs_layout_passes=False, use_tc_tiling_on_sc=True),
    scratch_types=[
        pltpu.VMEM((2 * PER_W,), jnp.int32),
        pltpu.VMEM((2 * PER_W,), jnp.int32),
        pltpu.VMEM((LANES, 8, EMB), jnp.float32),
        pltpu.VMEM((LANES, 8, EMB), jnp.float32),
        pltpu.VMEM((LANES, 8, EMB), jnp.float32),
        pltpu.VMEM((LANES, 8, EMB), jnp.float32),
        pltpu.VMEM((LANES, 8, EMB), jnp.float32),
        pltpu.VMEM((LANES, 8, EMB), jnp.float32),
        pltpu.VMEM((PER_W,), jnp.float32),
        pltpu.VMEM_SHARED((NUM_SUBCORES, PER_W), jnp.float32),
        pltpu.SemaphoreType.DMA,
        pltpu.SemaphoreType.DMA,
    ],
)(_dots_body)


def _final_body(uidx_hbm, midx_hbm, dots_hbm, ub16_hbm, mb16_hbm,
                out_hbm, uidx_v, midx_v, uhi_v, mhi_v, dvals_v, ubrows,
                mbrows, out_v, sem):
    sid = lax.axis_index("s")
    co = lax.axis_index("c")
    wid = co * NUM_SUBCORES + sid
    pltpu.sync_copy(uidx_hbm.at[wid], uidx_v)
    pltpu.sync_copy(midx_hbm.at[wid], midx_v)
    pltpu.sync_copy(dots_hbm.at[wid], dvals_v)

    for t in range(PER_W // LANES):
        tsl = pl.ds(t * LANES, LANES)
        uhi_v[tsl] = lax.shift_right_logical(uidx_v[tsl], 4)
        mhi_v[tsl] = lax.shift_right_logical(midx_v[tsl], 4)
    copies = []
    for j in range(NCHUNK):
        isl = pl.ds(j * CHUNK, CHUNK)
        copies.append(pltpu.async_copy(ub16_hbm.at[uhi_v.at[isl]], ubrows.at[isl, :], sem))
        copies.append(pltpu.async_copy(mb16_hbm.at[mhi_v.at[isl]], mbrows.at[isl, :], sem))
    for c in copies:
        c.wait()

    lanes = lax.iota(jnp.int32, LANES)

    def group(g, carry):
        ib = g * LANES
        rvec = ib + lanes
        uld = uidx_v[pl.ds(ib, LANES)]
        mld = midx_v[pl.ds(ib, LANES)]
        dv = dvals_v[pl.ds(ib, LANES)]
        ub = plsc.load_gather(ubrows, [rvec, uld & 15])
        mb = plsc.load_gather(mbrows, [rvec, mld & 15])
        res = dv + ub + mb + jnp.float32(GLOBAL_MEAN)
        plsc.store_scatter(out_v, [ib + lanes], res)
        return carry

    lax.fori_loop(0, PER_W // LANES, group, 0)
    pltpu.sync_copy(out_v, out_hbm.at[wid])


_final_call = functools.partial(
    pl.kernel,
    out_type=jax.ShapeDtypeStruct((NW, PER_W), jnp.float32),
    mesh=plsc.VectorSubcoreMesh(core_axis_name="c", subcore_axis_name="s"),
    compiler_params=pltpu.CompilerParams(
        needs_layout_passes=False, use_tc_tiling_on_sc=False),
    scratch_types=[
        pltpu.VMEM((PER_W,), jnp.int32),
        pltpu.VMEM((PER_W,), jnp.int32),
        pltpu.VMEM((PER_W,), jnp.int32),
        pltpu.VMEM((PER_W,), jnp.int32),
        pltpu.VMEM((PER_W,), jnp.float32),
        pltpu.VMEM((PER_W, LANES), jnp.float32),
        pltpu.VMEM((PER_W, LANES), jnp.float32),
        pltpu.VMEM((PER_W,), jnp.float32),
        pltpu.SemaphoreType.DMA,
    ],
)(_final_body)


def kernel(user_idx, movie_idx, user_emb, movie_emb, user_bias, movie_bias):
    uidx = user_idx.astype(jnp.int32)
    midx = movie_idx.astype(jnp.int32)
    uemb3 = user_emb.reshape(-1, 8, EMB)
    memb3 = movie_emb.reshape(-1, 8, EMB)
    dots = _dots_call(uidx, midx, uemb3, memb3)
    out = _final_call(uidx.reshape(NW, PER_W), midx.reshape(NW, PER_W),
                      dots, user_bias.reshape(-1, LANES),
                      movie_bias.reshape(-1, LANES))
    return out.reshape(BATCH)
